# baseline (speedup 1.0000x reference)
import jax
import jax.numpy as jnp
from jax import lax
from jax.experimental import pallas as pl
from jax.experimental.pallas import tpu as pltpu
from jax.experimental.pallas import tpu_sc as plsc

D0, H, W = 21, 400, 352
D1 = (D0 - 3) // 2 + 1
D2 = (D1 - 3) // 2 + 1
HW = H * W
CELLS0, CELLS1, CELLS2 = D0 * HW, D1 * HW, D2 * HW

NC, NSUB = 2, 16
NW = NC * NSUB
CH = 128
GRP = 512
MSET = 65536

GA0 = NSUB * 3 * MSET
GA1 = NSUB * 2 * MSET
GA2 = NSUB * 1 * MSET
DP = 288 * 2048

SUBM_TAPS = tuple((dz, dy, dx) for dz in (-1, 0, 1) for dy in (-1, 0, 1)
                  for dx in (-1, 0, 1))

_SCPARAMS = pltpu.CompilerParams(use_tc_tiling_on_sc=False)


def _ceil_to(x, m):
    return (x + m - 1) // m * m


def _mesh2():
    return plsc.VectorSubcoreMesh(core_axis_name="c", subcore_axis_name="s")


def _mesh1():
    return plsc.VectorSubcoreMesh(core_axis_name="c", subcore_axis_name="s",
                                  num_cores=1)


def _build_grids(n, NP0):
    NS1 = 2 * NP0
    S = n
    dump0, dump1, dump2 = GA0 - 8, GA1 - 8, GA2 - 8
    gpt = NP0 // (NSUB * GRP)

    def body(zyx, g0, g1, g2, zoc, mbuf, zyxv, zocv, ib0, vb0, ib1, vb1,
             ib2, vb2, sem):
        wid = lax.axis_index("s")

        @pl.loop(0, MSET // 16)
        def _fill(i):
            mbuf[pl.ds(i * 16, 16)] = jnp.full((16,), S, jnp.int32)

        mdescs = []
        for grid, copies in ((g0, 3), (g1, 2), (g2, 1)):
            for j in range(copies):
                off = (wid * copies + j) * MSET
                mdescs.append(pltpu.async_copy(
                    mbuf, grid.at[pl.ds(pl.multiple_of(off, MSET), MSET)],
                    sem))
        for d in mdescs:
            d.wait()

        plsc.subcore_barrier()

        @pl.loop(0, gpt)
        def _group(gi):
            gb = wid * gpt + gi
            gbase = pl.multiple_of(gb * GRP, GRP)
            pltpu.sync_copy(zyx.at[gb], zyxv)
            for v in range(GRP // 16):
                sl5 = pl.ds(v * 16, 16)
                z = zyxv[0, sl5]
                yw = zyxv[1, sl5] * W + zyxv[2, sl5]
                lane = gbase + v * 16 + lax.iota(jnp.int32, 16)
                rv = z >= 0
                ib0[sl5] = jnp.where(rv, z * HW + yw, dump0)
                vb0[sl5] = lane
                zoa = z >> 1
                va = rv & (zoa < D1)
                zob = zoa - 1
                vbm = rv & ((z & 1) == 0) & (zob >= 0)
                zoa_m = jnp.where(va, zoa, -1)
                zob_m = jnp.where(vbm, zob, -1)
                zocv[0, sl5] = zoa_m
                zocv[1, sl5] = zob_m
                zocv[6, sl5] = yw
                ib1[sl5] = jnp.where(va, zoa * HW + yw, dump1)
                vb1[sl5] = lane
                ib1[pl.ds(GRP + v * 16, 16)] = jnp.where(
                    vbm, zob * HW + yw, dump1)
                vb1[pl.ds(GRP + v * 16, 16)] = NP0 + lane
                for w1, zo1v in ((0, zoa_m), (1, zob_m)):
                    z2a = zo1v >> 1
                    va2 = (zo1v >= 0) & (z2a < D2)
                    z2b = z2a - 1
                    vb2m = (zo1v >= 0) & ((zo1v & 1) == 0) & (z2b >= 0)
                    for w2, z2, vld in ((0, z2a, va2), (1, z2b, vb2m)):
                        r = w2 * 2 + w1
                        off = w2 * NS1 + w1 * NP0
                        sl2 = pl.ds(r * GRP + v * 16, 16)
                        ib2[sl2] = jnp.where(vld, z2 * HW + yw, dump2)
                        vb2[sl2] = off + lane
                        zocv[2 + r, sl5] = jnp.where(vld, z2, -1)
            pltpu.sync_copy(zocv, zoc.at[gb])
            descs = [pltpu.async_copy(vb0, g0.at[ib0], sem),
                     pltpu.async_copy(vb1, g1.at[ib1], sem),
                     pltpu.async_copy(vb2, g2.at[ib2], sem)]
            for d in descs:
                d.wait()

    return pl.kernel(
        body,
        out_type=[
            jax.ShapeDtypeStruct((GA0,), jnp.int32),
            jax.ShapeDtypeStruct((GA1,), jnp.int32),
            jax.ShapeDtypeStruct((GA2,), jnp.int32),
            jax.ShapeDtypeStruct((NP0 // GRP, 7, GRP), jnp.int32),
        ],
        mesh=_mesh1(),
        scratch_types=[
            pltpu.VMEM((MSET,), jnp.int32),
            pltpu.VMEM((3, GRP), jnp.int32),
            pltpu.VMEM((7, GRP), jnp.int32),
            pltpu.VMEM((GRP,), jnp.int32),
            pltpu.VMEM((GRP,), jnp.int32),
            pltpu.VMEM((2 * GRP,), jnp.int32),
            pltpu.VMEM((2 * GRP,), jnp.int32),
            pltpu.VMEM((4 * GRP,), jnp.int32),
            pltpu.VMEM((4 * GRP,), jnp.int32),
            pltpu.SemaphoreType.DMA,
        ],
        compiler_params=_SCPARAMS,
        interpret=False,
    )


def _make_idx_s0(NP0):
    gpt = NP0 // (NW * GRP)
    sent = CELLS0

    def body(grid, zyx, ridx, zyxv, ib, gv, sem):
        wid = lax.axis_index("s") * NC + lax.axis_index("c")

        @pl.loop(0, gpt)
        def _group(gi):
            gb = wid * gpt + gi
            pltpu.sync_copy(zyx.at[gb], zyxv)
            for hc in range(8):
                for v in range(4):
                    sl5 = pl.ds(hc * 64 + v * 16, 16)
                    z = zyxv[0, sl5]
                    y = zyxv[1, sl5]
                    x = zyxv[2, sl5]
                    rv = z >= 0
                    fl0 = z * HW + y * W + x
                    mz = {dz: rv & (z + dz >= 0) & (z + dz < D0)
                          for dz in (-1, 0, 1)}
                    my = {dy: (y + dy >= 0) & (y + dy < H)
                          for dy in (-1, 0, 1)}
                    mx = {dx: (x + dx >= 0) & (x + dx < W)
                          for dx in (-1, 0, 1)}
                    for k, (dz, dy, dx) in enumerate(SUBM_TAPS):
                        m = mz[dz] & my[dy] & mx[dx]
                        ib[pl.ds(k * 64 + v * 16, 16)] = jnp.where(
                            m, fl0 + ((dz * H + dy) * W + dx), sent)
                pltpu.async_copy(grid.at[ib], gv, sem).wait()
                pltpu.sync_copy(gv, ridx.at[gb * 8 + hc])

    return pl.kernel(
        body,
        out_type=jax.ShapeDtypeStruct((NP0 // 64, 27 * 64), jnp.int32),
        mesh=_mesh2(),
        scratch_types=[
            pltpu.VMEM((3, GRP), jnp.int32),
            pltpu.VMEM((27 * 64,), jnp.int32),
            pltpu.VMEM((27 * 64,), jnp.int32),
            pltpu.SemaphoreType.DMA,
        ],
        compiler_params=_SCPARAMS,
        interpret=False,
    )


def _make_idx_s1(NP0):
    NS1 = 2 * NP0
    gpt = NS1 // (NW * GRP)
    ngh = NP0 // GRP
    sent = CELLS1

    def body(grid, zyx, zoc, ridx, zyxv, zv, ib, gv, sem):
        wid = lax.axis_index("s") * NC + lax.axis_index("c")

        @pl.loop(0, gpt)
        def _group(gi):
            gb = wid * gpt + gi
            w1 = gb // ngh
            hg = gb % ngh
            pltpu.sync_copy(zyx.at[hg], zyxv)
            pltpu.sync_copy(zoc.at[hg, w1], zv)
            for c in range(4):
                for v in range(8):
                    sl5 = pl.ds(c * CH + v * 16, 16)
                    z = zv[sl5]
                    y = zyxv[1, sl5]
                    x = zyxv[2, sl5]
                    rv = z >= 0
                    fl0 = z * HW + y * W + x
                    mz = {dz: rv & (z + dz >= 0) & (z + dz < D1)
                          for dz in (-1, 0, 1)}
                    my = {dy: (y + dy >= 0) & (y + dy < H)
                          for dy in (-1, 0, 1)}
                    mx = {dx: (x + dx >= 0) & (x + dx < W)
                          for dx in (-1, 0, 1)}
                    for k, (dz, dy, dx) in enumerate(SUBM_TAPS):
                        m = mz[dz] & my[dy] & mx[dx]
                        ib[pl.ds(k * CH + v * 16, 16)] = jnp.where(
                            m, fl0 + ((dz * H + dy) * W + dx), sent)
                pltpu.async_copy(grid.at[ib], gv, sem).wait()
                pltpu.sync_copy(gv, ridx.at[gb * 4 + c])

    return pl.kernel(
        body,
        out_type=jax.ShapeDtypeStruct((NS1 // CH, 27 * CH), jnp.int32),
        mesh=_mesh2(),
        scratch_types=[
            pltpu.VMEM((3, GRP), jnp.int32),
            pltpu.VMEM((GRP,), jnp.int32),
            pltpu.VMEM((27 * CH,), jnp.int32),
            pltpu.VMEM((27 * CH,), jnp.int32),
            pltpu.SemaphoreType.DMA,
        ],
        compiler_params=_SCPARAMS,
        interpret=False,
    )


def _make_idx_conv(NR, NP0, d_in, sent, stage2):
    gpt = NR // (NW * GRP)
    ngh = NP0 // GRP

    def body(grid, zoc, ridx, zv, ywv, ib, gv, sem):
        wid = lax.axis_index("s") * NC + lax.axis_index("c")

        @pl.loop(0, gpt)
        def _group(gi):
            gb = wid * gpt + gi
            if stage2:
                w2 = gb // (2 * ngh)
                rem = gb % (2 * ngh)
                w1 = rem // ngh
                hg = rem % ngh
                zrow = 2 + 2 * w2 + w1
            else:
                w1 = gb // ngh
                hg = gb % ngh
                zrow = w1
            pltpu.sync_copy(zoc.at[hg, zrow], zv)
            pltpu.sync_copy(zoc.at[hg, 6], ywv)
            for c in range(4):
                for v in range(8):
                    sl5 = pl.ds(c * CH + v * 16, 16)
                    z = zv[sl5]
                    yw = ywv[sl5]
                    rv = z >= 0
                    fl0 = (z * 2) * HW + yw
                    for kd in range(3):
                        m = rv & (z * 2 + kd < d_in)
                        ib[pl.ds((c * 3 + kd) * CH + v * 16, 16)] = (
                            jnp.where(m, fl0 + kd * HW, sent))
            pltpu.async_copy(grid.at[ib], gv, sem).wait()
            pltpu.sync_copy(gv, ridx.at[gb])

    return pl.kernel(
        body,
        out_type=jax.ShapeDtypeStruct((NR // GRP, 12 * CH), jnp.int32),
        mesh=_mesh2(),
        scratch_types=[
            pltpu.VMEM((GRP,), jnp.int32),
            pltpu.VMEM((GRP,), jnp.int32),
            pltpu.VMEM((12 * CH,), jnp.int32),
            pltpu.VMEM((12 * CH,), jnp.int32),
            pltpu.SemaphoreType.DMA,
        ],
        compiler_params=_SCPARAMS,
        interpret=False,
    )


def _make_rowgather(NITER, KCH, C):
    ipt = NITER // NW

    def body(ridx, xsrc, G, ib, gbuf, sem):
        wid = lax.axis_index("s") * NC + lax.axis_index("c")

        @pl.loop(0, ipt)
        def _iter(it):
            gidx = wid * ipt + it
            pltpu.sync_copy(ridx.at[gidx], ib)
            pltpu.async_copy(xsrc.at[ib], gbuf, sem).wait()
            pltpu.sync_copy(gbuf, G.at[gidx])

    return pl.kernel(
        body,
        out_type=jax.ShapeDtypeStruct((NITER, KCH, C), jnp.float32),
        mesh=_mesh2(),
        scratch_types=[
            pltpu.VMEM((KCH,), jnp.int32),
            pltpu.VMEM((KCH, C), jnp.float32),
            pltpu.SemaphoreType.DMA,
        ],
        compiler_params=_SCPARAMS,
        interpret=False,
    )


def _mm_relu_subm(G3, Wall, CBI, CHM):
    NITER, KCH, C = G3.shape
    K = KCH // CHM
    BR = CBI * CHM
    NR = NITER * CHM
    CO = Wall.shape[1]

    def body(x_ref, w_ref, o_ref):
        xb = x_ref[...]
        parts = [xb[:, k * CHM:(k + 1) * CHM, :].reshape(BR, C)
                 for k in range(K)]
        x = jnp.concatenate(parts, axis=1)
        o_ref[...] = jax.nn.relu(
            jnp.dot(x, w_ref[...], preferred_element_type=jnp.float32))

    return pl.pallas_call(
        body,
        grid=(NITER // CBI,),
        in_specs=[
            pl.BlockSpec((CBI, KCH, C), lambda r: (r, 0, 0)),
            pl.BlockSpec((K * C, CO), lambda r: (0, 0)),
        ],
        out_specs=pl.BlockSpec((BR, CO), lambda r: (r, 0)),
        out_shape=jax.ShapeDtypeStruct((NR, CO), jnp.float32),
    )(G3, Wall)


def _mm_relu_conv(G3, Wall, CBI):
    NITER, KCH, C = G3.shape
    BR = CBI * 4 * CH
    NR = NITER * 4 * CH
    CO = Wall.shape[1]

    def body(x_ref, w_ref, o_ref):
        xb = x_ref[...]
        parts = []
        for k in range(3):
            sub = jnp.concatenate(
                [xb[:, (3 * s + k) * CH:(3 * s + k + 1) * CH, :][:, None]
                 for s in range(4)], axis=1)
            parts.append(sub.reshape(BR, C))
        x = jnp.concatenate(parts, axis=1)
        o_ref[...] = jax.nn.relu(
            jnp.dot(x, w_ref[...], preferred_element_type=jnp.float32))

    return pl.pallas_call(
        body,
        grid=(NITER // CBI,),
        in_specs=[
            pl.BlockSpec((CBI, KCH, C), lambda r: (r, 0, 0)),
            pl.BlockSpec((3 * C, CO), lambda r: (0, 0)),
        ],
        out_specs=pl.BlockSpec((BR, CO), lambda r: (r, 0)),
        out_shape=jax.ShapeDtypeStruct((NR, CO), jnp.float32),
    )(G3, Wall)


def _transpose_tc(dense3):
    d2, hw, co = dense3.shape
    BP = 1280

    def body(x_ref, o_ref):
        for z in range(d2):
            o_ref[:, z, :] = x_ref[z].T

    return pl.pallas_call(
        body,
        grid=(hw // BP,),
        in_specs=[pl.BlockSpec((d2, BP, co), lambda p: (0, p, 0))],
        out_specs=pl.BlockSpec((co, d2, BP), lambda p: (0, 0, p)),
        out_shape=jax.ShapeDtypeStruct((co, d2, hw), jnp.float32),
    )(dense3)


def kernel(voxel_features, coors, batch_size, W_subm0, W_conv1, W_subm1,
           W_subm2, W_conv2):
    n, C0 = voxel_features.shape
    NP0 = _ceil_to(n + 1, NW * GRP)
    NS1, NT2 = 2 * NP0, 4 * NP0

    zpad = jnp.full((NP0 - n,), -1, jnp.int32)
    opad = jnp.zeros((NP0 - n,), jnp.int32)
    z0 = jnp.concatenate([coors[:, 1].astype(jnp.int32), zpad])
    y0 = jnp.concatenate([coors[:, 2].astype(jnp.int32), opad])
    x0 = jnp.concatenate([coors[:, 3].astype(jnp.int32), opad])
    zyx = jnp.stack([z0, y0, x0]).reshape(3, NP0 // GRP, GRP).transpose(
        1, 0, 2)
    feats = jnp.zeros((NP0, C0), jnp.float32).at[:n].set(voxel_features)

    g0, g1, g2, zoc = _build_grids(n, NP0)(zyx)

    ridx0 = _make_idx_s0(NP0)(g0, zyx)
    G0 = _make_rowgather(NP0 // 64, 27 * 64, C0)(ridx0, feats)
    x_0 = _mm_relu_subm(G0, W_subm0.reshape(27 * C0, 32), 16, 64)

    rc1 = _make_idx_conv(NS1, NP0, D0, CELLS0, False)(g0, zoc)
    Gc1 = _make_rowgather(NS1 // GRP, 12 * CH, 32)(rc1, x_0)
    x_1 = _mm_relu_conv(Gc1, W_conv1.reshape(96, 32), 2)

    ridx1 = _make_idx_s1(NP0)(g1, zyx, zoc)
    G1a = _make_rowgather(NS1 // CH, 27 * CH, 32)(ridx1, x_1)
    x_1a = _mm_relu_subm(G1a, W_subm1.reshape(27 * 32, 32), 8, CH)
    G1b = _make_rowgather(NS1 // CH, 27 * CH, 32)(ridx1, x_1a)
    x_1b = _mm_relu_subm(G1b, W_subm2.reshape(27 * 32, 32), 8, CH)

    rc2 = _make_idx_conv(NT2, NP0, D1, CELLS1, True)(g1, zoc)
    Gc2 = _make_rowgather(NT2 // GRP, 12 * CH, 32)(rc2, x_1b)
    x_2 = _mm_relu_conv(Gc2, W_conv2.reshape(96, 32), 2)

    g2v = g2.reshape(GA2 // 2048, 2048)[:DP // 2048]
    dense4 = _make_rowgather(DP // 2048, 2048, 32)(g2v, x_2)
    dense = dense4.reshape(DP, 32)
    out3 = _transpose_tc(dense[:CELLS2].reshape(D2, HW, 32))
    return out3.reshape(1, 32 * D2, H, W)
